# baseline (speedup 1.0000x reference)
import functools

import jax
import jax.numpy as jnp
from jax import lax
from jax.experimental import pallas as pl
from jax.experimental.pallas import tpu as pltpu
from jax.experimental.pallas import tpu_sc as plsc

ALPHA = 0.2
N = 100000
D = 128
EMB = 128
C = 40
B = 16384
S1 = 5
S2 = 10

NC = 2
NS = 16
L = 16
NW = NC * NS

BC1 = 64
CPW1 = 50
G1 = CPW1 // 2
NPAD = NW * CPW1 * BC1
R1 = S1 + 1

BC2 = 32
CPW2 = B // (NW * BC2)
G2 = CPW2 // 2

_MESH = plsc.VectorSubcoreMesh(
    core_axis_name="c", subcore_axis_name="s", num_cores=NC, num_subcores=NS)


@functools.partial(
    pl.kernel,
    out_type=jax.ShapeDtypeStruct((NPAD, D), jnp.float32),
    mesh=_MESH,
    scratch_types=[
        pltpu.VMEM((BC1 * R1,), jnp.int32),
        pltpu.VMEM((BC1 * R1,), jnp.int32),
        pltpu.VMEM((BC1 * R1, D), jnp.float32),
        pltpu.VMEM((BC1 * R1, D), jnp.float32),
        pltpu.VMEM((BC1, D), jnp.float32),
        pltpu.SemaphoreType.DMA,
        pltpu.SemaphoreType.DMA,
    ],
)
def _agg1(feat_hbm, idx_hbm, out_hbm,
          idx0_v, idx1_v, rows0_v, rows1_v, acc_v, sem0, sem1):
    wid = lax.axis_index("s") * NC + lax.axis_index("c")

    def fetch(c, idx_v, rows_v, sem):
        base = (wid * CPW1 + c) * BC1
        pltpu.sync_copy(idx_hbm.at[pl.ds(base * R1, BC1 * R1)], idx_v)
        pltpu.async_copy(feat_hbm.at[idx_v], rows_v, sem).wait()

    def consume(c, rows_v, sem):
        base = (wid * CPW1 + c) * BC1

        def node(i, cc):
            def dcol(d, ccc):
                col = pl.ds(d * L, L)
                s = rows_v[R1 * i, col]
                for j in range(1, R1):
                    s = s + rows_v[R1 * i + j, col]
                acc_v[i, col] = s
                return ccc
            return lax.fori_loop(0, D // L, dcol, cc)

        lax.fori_loop(0, BC1, node, 0)
        pltpu.sync_copy(acc_v, out_hbm.at[pl.ds(base, BC1)])

    def chunk(c, carry):
        fetch(c, idx0_v, rows0_v, sem0)
        consume(c, rows0_v, sem0)
        return carry

    lax.fori_loop(0, CPW1, chunk, 0)


@functools.partial(
    pl.kernel,
    out_type=jax.ShapeDtypeStruct((B, EMB), jnp.float32),
    mesh=_MESH,
    scratch_types=[
        pltpu.VMEM((BC2,), jnp.int32),
        pltpu.VMEM((BC2,), jnp.int32),
        pltpu.VMEM((BC2 * S2,), jnp.int32),
        pltpu.VMEM((BC2 * S2,), jnp.int32),
        pltpu.VMEM((BC2 * S2,), jnp.int32),
        pltpu.VMEM((BC2 * S2,), jnp.int32),
        pltpu.VMEM((BC2 * S2, EMB), jnp.float32),
        pltpu.VMEM((BC2 * S2, EMB), jnp.float32),
        pltpu.VMEM((BC2, EMB), jnp.float32),
        pltpu.VMEM((BC2, EMB), jnp.float32),
        pltpu.VMEM((BC2, EMB), jnp.float32),
        pltpu.SemaphoreType.DMA,
        pltpu.SemaphoreType.DMA,
        pltpu.SemaphoreType.DMA,
        pltpu.SemaphoreType.DMA,
        pltpu.SemaphoreType.DMA,
        pltpu.SemaphoreType.DMA,
    ],
)
def _agg2(nodes_hbm, pos_hbm, neigh2f_hbm, h1_hbm, out_hbm,
          nodes0_v, nodes1_v, pos0_v, pos1_v, nidx0_v, nidx1_v,
          rows0_v, rows1_v, self0_v, self1_v, acc_v,
          semA0, semA1, semB0, semB1, semC0, semC1):
    wid = lax.axis_index("s") * NC + lax.axis_index("c")

    def s1(c, nodes_v, pos_v, nidx_v, semA):
        nbase = (wid * CPW2 + c) * BC2
        pltpu.sync_copy(nodes_hbm.at[pl.ds(nbase, BC2)], nodes_v)
        pltpu.sync_copy(pos_hbm.at[pl.ds(nbase * S2, BC2 * S2)], pos_v)
        pltpu.async_copy(neigh2f_hbm.at[pos_v], nidx_v, semA)

    def s2(nodes_v, nidx_v, rows_v, self_v, semA, semB, semC):
        pltpu.make_async_copy(neigh2f_hbm.at[pl.ds(0, BC2 * S2)], nidx_v,
                              semA).wait()
        pltpu.async_copy(h1_hbm.at[nidx_v], rows_v, semB)
        pltpu.async_copy(h1_hbm.at[nodes_v], self_v, semC)

    def s3(c, rows_v, self_v, semB, semC):
        nbase = (wid * CPW2 + c) * BC2
        pltpu.make_async_copy(h1_hbm.at[pl.ds(0, BC2 * S2)], rows_v,
                              semB).wait()
        pltpu.make_async_copy(h1_hbm.at[pl.ds(0, BC2)], self_v, semC).wait()

        def node(i, cc):
            def dcol(d, ccc):
                col = pl.ds(d * L, L)
                s = self_v[i, col]
                for j in range(S2):
                    s = s + rows_v[S2 * i + j, col]
                acc_v[i, col] = s
                return ccc
            return lax.fori_loop(0, EMB // L, dcol, cc)

        lax.fori_loop(0, BC2, node, 0)
        pltpu.sync_copy(acc_v, out_hbm.at[pl.ds(nbase, BC2)])

    def chunk(c, carry):
        s1(c, nodes0_v, pos0_v, nidx0_v, semA0)
        s2(nodes0_v, nidx0_v, rows0_v, self0_v, semA0, semB0, semC0)
        s3(c, rows0_v, self0_v, semB0, semC0)
        return carry

    lax.fori_loop(0, CPW2, chunk, 0)


BLK1 = 2048


def _mm1_body(x_ref, w_ref, o_ref):
    y = jnp.dot(x_ref[...], w_ref[...], preferred_element_type=jnp.float32)
    o_ref[...] = jnp.maximum(y, ALPHA * y)


def _mm2_body(x_ref, w2_ref, wc_ref, o_ref):
    y = jnp.dot(x_ref[...], w2_ref[...], preferred_element_type=jnp.float32)
    h = jnp.maximum(y, ALPHA * y)
    o_ref[...] = jnp.dot(h, wc_ref[...], preferred_element_type=jnp.float32)


_tc1 = pl.pallas_call(
    _mm1_body,
    grid=(NPAD // BLK1,),
    in_specs=[
        pl.BlockSpec((BLK1, D), lambda i: (i, 0)),
        pl.BlockSpec((D, EMB), lambda i: (0, 0)),
    ],
    out_specs=pl.BlockSpec((BLK1, EMB), lambda i: (i, 0)),
    out_shape=jax.ShapeDtypeStruct((NPAD, EMB), jnp.float32),
)

_tc2 = pl.pallas_call(
    _mm2_body,
    grid=(B // BLK1,),
    in_specs=[
        pl.BlockSpec((BLK1, EMB), lambda i: (i, 0)),
        pl.BlockSpec((EMB, EMB), lambda i: (0, 0)),
        pl.BlockSpec((EMB, C), lambda i: (0, 0)),
    ],
    out_specs=pl.BlockSpec((BLK1, C), lambda i: (i, 0)),
    out_shape=jax.ShapeDtypeStruct((B, C), jnp.float32),
)


def kernel(nodes, neigh_l1, neigh_l2, features, W1, W2, class_weight):
    idx1 = jnp.concatenate(
        [neigh_l1, jnp.arange(N, dtype=jnp.int32)[:, None]], axis=1)
    idx1 = jnp.pad(idx1.reshape(-1), (0, (NPAD + BC1 - N) * R1))

    pos = (nodes[:, None] * S2 + jnp.arange(S2, dtype=jnp.int32)[None, :])
    pos = jnp.pad(pos.reshape(-1), (0, 2 * BC2 * S2))
    nodes_p = jnp.pad(nodes, (0, 2 * BC2))

    sum1 = _agg1(features, idx1)
    h1 = _tc1(sum1, W1 * (1.0 / R1))
    sum2 = _agg2(nodes_p, pos, neigh_l2.reshape(-1), h1)
    return _tc2(sum2, W2 * (1.0 / (S2 + 1)), class_weight.T)


# speedup vs baseline: 1.0997x; 1.0997x over previous
import functools

import jax
import jax.numpy as jnp
from jax import lax
from jax.experimental import pallas as pl
from jax.experimental.pallas import tpu as pltpu
from jax.experimental.pallas import tpu_sc as plsc

ALPHA = 0.2
N = 100000
D = 128
EMB = 128
C = 40
B = 16384
S1 = 5
S2 = 10

NC = 2
NS = 16
L = 16
NW = NC * NS

BC1 = 64
CPW1 = 50
G1 = CPW1 // 2
NPAD = NW * CPW1 * BC1
R1 = S1 + 1

BC2 = 32
CPW2 = B // (NW * BC2)
G2 = CPW2 // 2

_MESH = plsc.VectorSubcoreMesh(
    core_axis_name="c", subcore_axis_name="s", num_cores=NC, num_subcores=NS)


@functools.partial(
    pl.kernel,
    out_type=jax.ShapeDtypeStruct((NPAD, D), jnp.float32),
    mesh=_MESH,
    scratch_types=[
        pltpu.VMEM((BC1 * R1,), jnp.int32),
        pltpu.VMEM((BC1 * R1,), jnp.int32),
        pltpu.VMEM((BC1 * R1, D), jnp.float32),
        pltpu.VMEM((BC1 * R1, D), jnp.float32),
        pltpu.VMEM((BC1, D), jnp.float32),
        pltpu.SemaphoreType.DMA,
        pltpu.SemaphoreType.DMA,
    ],
)
def _agg1(feat_hbm, idx_hbm, out_hbm,
          idx0_v, idx1_v, rows0_v, rows1_v, acc_v, sem0, sem1):
    wid = lax.axis_index("s") * NC + lax.axis_index("c")

    def fetch(c, idx_v, rows_v, sem):
        base = (c * NW + wid) * BC1
        pltpu.sync_copy(idx_hbm.at[pl.ds(base * R1, BC1 * R1)], idx_v)
        pltpu.async_copy(feat_hbm.at[idx_v], rows_v, sem).wait()

    def consume(c, rows_v, sem):
        base = (c * NW + wid) * BC1

        def node(i, cc):
            def dcol(d, ccc):
                col = pl.ds(d * L, L)
                s = rows_v[R1 * i, col]
                for j in range(1, R1):
                    s = s + rows_v[R1 * i + j, col]
                acc_v[i, col] = s
                return ccc
            return lax.fori_loop(0, D // L, dcol, cc)

        lax.fori_loop(0, BC1, node, 0)
        pltpu.sync_copy(acc_v, out_hbm.at[pl.ds(base, BC1)])

    def chunk(c, carry):
        fetch(c, idx0_v, rows0_v, sem0)
        consume(c, rows0_v, sem0)
        return carry

    lax.fori_loop(0, CPW1, chunk, 0)


@functools.partial(
    pl.kernel,
    out_type=jax.ShapeDtypeStruct((B, EMB), jnp.float32),
    mesh=_MESH,
    scratch_types=[
        pltpu.VMEM((BC2,), jnp.int32),
        pltpu.VMEM((BC2,), jnp.int32),
        pltpu.VMEM((BC2 * S2,), jnp.int32),
        pltpu.VMEM((BC2 * S2,), jnp.int32),
        pltpu.VMEM((BC2 * S2,), jnp.int32),
        pltpu.VMEM((BC2 * S2,), jnp.int32),
        pltpu.VMEM((BC2 * S2, EMB), jnp.float32),
        pltpu.VMEM((BC2 * S2, EMB), jnp.float32),
        pltpu.VMEM((BC2, EMB), jnp.float32),
        pltpu.VMEM((BC2, EMB), jnp.float32),
        pltpu.VMEM((BC2, EMB), jnp.float32),
        pltpu.SemaphoreType.DMA,
        pltpu.SemaphoreType.DMA,
        pltpu.SemaphoreType.DMA,
        pltpu.SemaphoreType.DMA,
        pltpu.SemaphoreType.DMA,
        pltpu.SemaphoreType.DMA,
    ],
)
def _agg2(nodes_hbm, pos_hbm, neigh2f_hbm, h1_hbm, out_hbm,
          nodes0_v, nodes1_v, pos0_v, pos1_v, nidx0_v, nidx1_v,
          rows0_v, rows1_v, self0_v, self1_v, acc_v,
          semA0, semA1, semB0, semB1, semC0, semC1):
    wid = lax.axis_index("s") * NC + lax.axis_index("c")

    def s1(c, nodes_v, pos_v, nidx_v, semA):
        nbase = (c * NW + wid) * BC2
        pltpu.sync_copy(nodes_hbm.at[pl.ds(nbase, BC2)], nodes_v)
        pltpu.sync_copy(pos_hbm.at[pl.ds(nbase * S2, BC2 * S2)], pos_v)
        pltpu.async_copy(neigh2f_hbm.at[pos_v], nidx_v, semA)

    def s2(nodes_v, nidx_v, rows_v, self_v, semA, semB, semC):
        pltpu.make_async_copy(neigh2f_hbm.at[pl.ds(0, BC2 * S2)], nidx_v,
                              semA).wait()
        pltpu.async_copy(h1_hbm.at[nidx_v], rows_v, semB)
        pltpu.async_copy(h1_hbm.at[nodes_v], self_v, semC)

    def s3(c, rows_v, self_v, semB, semC):
        nbase = (c * NW + wid) * BC2
        pltpu.make_async_copy(h1_hbm.at[pl.ds(0, BC2 * S2)], rows_v,
                              semB).wait()
        pltpu.make_async_copy(h1_hbm.at[pl.ds(0, BC2)], self_v, semC).wait()

        def node(i, cc):
            def dcol(d, ccc):
                col = pl.ds(d * L, L)
                s = self_v[i, col]
                for j in range(S2):
                    s = s + rows_v[S2 * i + j, col]
                acc_v[i, col] = s
                return ccc
            return lax.fori_loop(0, EMB // L, dcol, cc)

        lax.fori_loop(0, BC2, node, 0)
        pltpu.sync_copy(acc_v, out_hbm.at[pl.ds(nbase, BC2)])

    def chunk(c, carry):
        s1(c, nodes0_v, pos0_v, nidx0_v, semA0)
        s2(nodes0_v, nidx0_v, rows0_v, self0_v, semA0, semB0, semC0)
        s3(c, rows0_v, self0_v, semB0, semC0)
        return carry

    lax.fori_loop(0, CPW2, chunk, 0)


BLK1 = 2048


def _mm1_body(x_ref, w_ref, o_ref):
    y = jnp.dot(x_ref[...], w_ref[...], preferred_element_type=jnp.float32)
    o_ref[...] = jnp.maximum(y, ALPHA * y)


def _mm2_body(x_ref, w2_ref, wc_ref, o_ref):
    y = jnp.dot(x_ref[...], w2_ref[...], preferred_element_type=jnp.float32)
    h = jnp.maximum(y, ALPHA * y)
    o_ref[...] = jnp.dot(h, wc_ref[...], preferred_element_type=jnp.float32)


_tc1 = pl.pallas_call(
    _mm1_body,
    grid=(NPAD // BLK1,),
    in_specs=[
        pl.BlockSpec((BLK1, D), lambda i: (i, 0)),
        pl.BlockSpec((D, EMB), lambda i: (0, 0)),
    ],
    out_specs=pl.BlockSpec((BLK1, EMB), lambda i: (i, 0)),
    out_shape=jax.ShapeDtypeStruct((NPAD, EMB), jnp.float32),
)

_tc2 = pl.pallas_call(
    _mm2_body,
    grid=(B // BLK1,),
    in_specs=[
        pl.BlockSpec((BLK1, EMB), lambda i: (i, 0)),
        pl.BlockSpec((EMB, EMB), lambda i: (0, 0)),
        pl.BlockSpec((EMB, C), lambda i: (0, 0)),
    ],
    out_specs=pl.BlockSpec((BLK1, C), lambda i: (i, 0)),
    out_shape=jax.ShapeDtypeStruct((B, C), jnp.float32),
)


def kernel(nodes, neigh_l1, neigh_l2, features, W1, W2, class_weight):
    idx1 = jnp.concatenate(
        [neigh_l1, jnp.arange(N, dtype=jnp.int32)[:, None]], axis=1)
    idx1 = jnp.pad(idx1.reshape(-1), (0, (NPAD + BC1 - N) * R1))

    pos = (nodes[:, None] * S2 + jnp.arange(S2, dtype=jnp.int32)[None, :])
    pos = jnp.pad(pos.reshape(-1), (0, 2 * BC2 * S2))
    nodes_p = jnp.pad(nodes, (0, 2 * BC2))

    sum1 = _agg1(features, idx1)
    h1 = _tc1(sum1, W1 * (1.0 / R1))
    sum2 = _agg2(nodes_p, pos, neigh_l2.reshape(-1), h1)
    return _tc2(sum2, W2 * (1.0 / (S2 + 1)), class_weight.T)


# speedup vs baseline: 1.7513x; 1.5926x over previous
import functools

import jax
import jax.numpy as jnp
from jax import lax
from jax.experimental import pallas as pl
from jax.experimental.pallas import tpu as pltpu
from jax.experimental.pallas import tpu_sc as plsc

ALPHA = 0.2
N = 100000
D = 128
EMB = 128
C = 40
B = 16384
S1 = 5
S2 = 10

NC = 2
NS = 16
L = 16
NW = NC * NS

BC1 = 64
CPW1 = 49
NPAD = NW * CPW1 * BC1
R1 = S1 + 1

BC2 = 32
CPW2 = B // (NW * BC2)

_MESH = plsc.VectorSubcoreMesh(
    core_axis_name="c", subcore_axis_name="s", num_cores=NC, num_subcores=NS)


@functools.partial(
    pl.kernel,
    out_type=jax.ShapeDtypeStruct((NPAD, D), jnp.float32),
    mesh=_MESH,
    scratch_types=[
        pltpu.VMEM((BC1 * R1,), jnp.int32),
        pltpu.VMEM((BC1 * R1, D), jnp.float32),
        pltpu.VMEM((BC1, D), jnp.float32),
        pltpu.SemaphoreType.DMA,
    ],
)
def _agg1(feat_hbm, idx_hbm, out_hbm, idx_v, rows_v, acc_v, sem):
    wid = lax.axis_index("s") * NC + lax.axis_index("c")

    def chunk(c, carry):
        base = (c * NW + wid) * BC1
        pltpu.sync_copy(idx_hbm.at[pl.ds(base * R1, BC1 * R1)], idx_v)
        pltpu.async_copy(feat_hbm.at[idx_v], rows_v, sem).wait()

        def node(i, cc):
            def dcol(d, ccc):
                col = pl.ds(d * L, L)
                s = rows_v[R1 * i, col]
                for j in range(1, R1):
                    s = s + rows_v[R1 * i + j, col]
                acc_v[i, col] = s
                return ccc
            return lax.fori_loop(0, D // L, dcol, cc)

        lax.fori_loop(0, BC1, node, 0)
        pltpu.sync_copy(acc_v, out_hbm.at[pl.ds(base, BC1)])
        return carry

    lax.fori_loop(0, CPW1, chunk, 0)


@functools.partial(
    pl.kernel,
    out_type=jax.ShapeDtypeStruct((B, EMB), jnp.float32),
    mesh=_MESH,
    scratch_types=[
        pltpu.VMEM((BC2,), jnp.int32),
        pltpu.VMEM((BC2 * S2,), jnp.int32),
        pltpu.VMEM((BC2 * S2,), jnp.int32),
        pltpu.VMEM((BC2 * S2, EMB), jnp.float32),
        pltpu.VMEM((BC2, EMB), jnp.float32),
        pltpu.VMEM((BC2, EMB), jnp.float32),
        pltpu.SemaphoreType.DMA,
        pltpu.SemaphoreType.DMA,
    ],
)
def _agg2(nodes_hbm, pos_hbm, neigh2f_hbm, h1_hbm, out_hbm,
          nodes_v, pos_v, nidx_v, rows_v, self_v, acc_v, sem_a, sem_b):
    wid = lax.axis_index("s") * NC + lax.axis_index("c")

    def chunk(c, carry):
        nbase = (c * NW + wid) * BC2
        pltpu.sync_copy(nodes_hbm.at[pl.ds(nbase, BC2)], nodes_v)
        pltpu.sync_copy(pos_hbm.at[pl.ds(nbase * S2, BC2 * S2)], pos_v)
        pltpu.async_copy(neigh2f_hbm.at[pos_v], nidx_v, sem_a).wait()

        cp_rows = pltpu.async_copy(h1_hbm.at[nidx_v], rows_v, sem_a)
        cp_self = pltpu.async_copy(h1_hbm.at[nodes_v], self_v, sem_b)
        cp_rows.wait()
        cp_self.wait()

        def node(i, cc):
            def dcol(d, ccc):
                col = pl.ds(d * L, L)
                s = self_v[i, col]
                for j in range(S2):
                    s = s + rows_v[S2 * i + j, col]
                acc_v[i, col] = s
                return ccc
            return lax.fori_loop(0, EMB // L, dcol, cc)

        lax.fori_loop(0, BC2, node, 0)
        pltpu.sync_copy(acc_v, out_hbm.at[pl.ds(nbase, BC2)])
        return carry

    lax.fori_loop(0, CPW2, chunk, 0)


BLK1 = 2048


def _mm1_body(x_ref, w_ref, o_ref):
    y = jnp.dot(x_ref[...], w_ref[...], preferred_element_type=jnp.float32)
    o_ref[...] = jnp.maximum(y, ALPHA * y)


def _mm2_body(x_ref, w2_ref, wc_ref, o_ref):
    y = jnp.dot(x_ref[...], w2_ref[...], preferred_element_type=jnp.float32)
    h = jnp.maximum(y, ALPHA * y)
    o_ref[...] = jnp.dot(h, wc_ref[...], preferred_element_type=jnp.float32)


_tc1 = pl.pallas_call(
    _mm1_body,
    grid=(NPAD // BLK1,),
    in_specs=[
        pl.BlockSpec((BLK1, D), lambda i: (i, 0)),
        pl.BlockSpec((D, EMB), lambda i: (0, 0)),
    ],
    out_specs=pl.BlockSpec((BLK1, EMB), lambda i: (i, 0)),
    out_shape=jax.ShapeDtypeStruct((NPAD, EMB), jnp.float32),
)

_tc2 = pl.pallas_call(
    _mm2_body,
    grid=(B // BLK1,),
    in_specs=[
        pl.BlockSpec((BLK1, EMB), lambda i: (i, 0)),
        pl.BlockSpec((EMB, EMB), lambda i: (0, 0)),
        pl.BlockSpec((EMB, C), lambda i: (0, 0)),
    ],
    out_specs=pl.BlockSpec((BLK1, C), lambda i: (i, 0)),
    out_shape=jax.ShapeDtypeStruct((B, C), jnp.float32),
)


def kernel(nodes, neigh_l1, neigh_l2, features, W1, W2, class_weight):
    idx1 = jnp.concatenate(
        [neigh_l1, jnp.arange(N, dtype=jnp.int32)[:, None]], axis=1)
    idx1 = jnp.pad(idx1.reshape(-1), (0, (NPAD - N) * R1))

    pos = (nodes[:, None] * S2 + jnp.arange(S2, dtype=jnp.int32)[None, :])
    pos = pos.reshape(-1)

    sum1 = _agg1(features, idx1)
    h1 = _tc1(sum1, W1 * (1.0 / R1))
    sum2 = _agg2(nodes, pos, neigh_l2.reshape(-1), h1)
    return _tc2(sum2, W2 * (1.0 / (S2 + 1)), class_weight.T)
